# baseline (speedup 1.0000x reference)
import jax
import jax.numpy as jnp
from jax import lax
from jax.experimental import pallas as pl
from jax.experimental.pallas import tpu as pltpu
from jax.experimental.pallas import tpu_sc as plsc

_B = 4
_T = 4096
_D = 1024
_TDW = _T * _D
_NW = 32
_R = 8
_CW = _R * _D
_NCH = _T // _NW // _R


def _sc_body(x1, pe1, out1, spe0, spe1, sxb0, sxb1, si0, si1, so0, so1):
    spe = [spe0, spe1]
    sxb = [sxb0, sxb1]
    si = [si0, si1]
    so = [so0, so1]

    wid = lax.axis_index("s") * 2 + lax.axis_index("c")
    base = wid * _NCH * _CW

    def in_copies(j, s):
        off = base + j * _CW
        cps = [pltpu.make_async_copy(pe1.at[pl.ds(off, _CW)], spe[s], si[s])]
        for b in range(_B):
            cps.append(pltpu.make_async_copy(
                x1.at[pl.ds(b * _TDW + off, _CW)], sxb[s].at[b], si[s]))
        return cps

    def out_copies(j, s):
        off = base + j * _CW
        return [pltpu.make_async_copy(
            sxb[s].at[b], out1.at[pl.ds(b * _TDW + off, _CW)], so[s])
            for b in range(_B)]

    for cp in in_copies(0, 0):
        cp.start()
    for j in range(_NCH):
        s = j & 1
        if j + 1 < _NCH:
            s2 = (j + 1) & 1
            if j - 1 >= 0:
                for cp in out_copies(j - 1, s2):
                    cp.wait()
            for cp in in_copies(j + 1, s2):
                cp.start()
        for cp in in_copies(j, s):
            cp.wait()

        @plsc.parallel_loop(0, _CW // 16, 1, unroll=8)
        def _(k):
            sl = pl.ds(k * 16, 16)
            pv = spe[s][sl]
            for b in range(_B):
                sxb[s][b, sl] = sxb[s][b, sl] + pv

        for cp in out_copies(j, s):
            cp.start()
    for j in (_NCH - 2, _NCH - 1):
        for cp in out_copies(j, j & 1):
            cp.wait()


def _sc_add(x1, pe1):
    return pl.kernel(
        _sc_body,
        out_type=jax.ShapeDtypeStruct((_B * _TDW,), jnp.float32),
        mesh=plsc.VectorSubcoreMesh(core_axis_name="c", subcore_axis_name="s"),
        scratch_types=[
            pltpu.VMEM((_CW,), jnp.float32),
            pltpu.VMEM((_CW,), jnp.float32),
            pltpu.VMEM((_B, _CW), jnp.float32),
            pltpu.VMEM((_B, _CW), jnp.float32),
            pltpu.SemaphoreType.DMA,
            pltpu.SemaphoreType.DMA,
            pltpu.SemaphoreType.DMA,
            pltpu.SemaphoreType.DMA,
        ],
    )(x1, pe1)


def kernel(x, pos_embed):
    B, T, D = x.shape
    out1 = _sc_add(x.reshape(-1), pos_embed.reshape(-1))
    return out1.reshape(B, T, D)


# speedup vs baseline: 2.9370x; 2.9370x over previous
import jax
import jax.numpy as jnp
from jax import lax
from jax.experimental import pallas as pl
from jax.experimental.pallas import tpu as pltpu
from jax.experimental.pallas import tpu_sc as plsc

_B = 4
_T = 4096
_D = 1024
_NW = 32
_R = 8
_NCH = _T // _NW // _R
_NV = _R * _D // 16


def _sc_body(x2, pe2, out2, spe0, spe1, sxb0, sxb1, si0, si1, so0, so1):
    spe = [spe0, spe1]
    sxb = [sxb0, sxb1]
    si = [si0, si1]
    so = [so0, so1]

    wid = lax.axis_index("s") * 2 + lax.axis_index("c")
    base = wid * _NCH * _R

    def in_copies(j, s):
        row = base + j * _R
        cps = [pltpu.make_async_copy(pe2.at[pl.ds(row, _R)], spe[s], si[s])]
        for b in range(_B):
            cps.append(pltpu.make_async_copy(
                x2.at[pl.ds(b * _T + row, _R)], sxb[s].at[b], si[s]))
        return cps

    def out_copies(j, s):
        row = base + j * _R
        return [pltpu.make_async_copy(
            sxb[s].at[b], out2.at[pl.ds(b * _T + row, _R)], so[s])
            for b in range(_B)]

    for cp in in_copies(0, 0):
        cp.start()
    for j in range(_NCH):
        s = j & 1
        if j + 1 < _NCH:
            s2 = (j + 1) & 1
            if j - 1 >= 0:
                for cp in out_copies(j - 1, s2):
                    cp.wait()
            for cp in in_copies(j + 1, s2):
                cp.start()
        for cp in in_copies(j, s):
            cp.wait()

        @plsc.parallel_loop(0, _NV, 1, unroll=8)
        def _(k):
            r = k // (_D // 16)
            sl = pl.ds((k % (_D // 16)) * 16, 16)
            pv = spe[s][r, sl]
            for b in range(_B):
                sxb[s][b, r, sl] = sxb[s][b, r, sl] + pv

        for cp in out_copies(j, s):
            cp.start()
    for j in (_NCH - 2, _NCH - 1):
        for cp in out_copies(j, j & 1):
            cp.wait()


def _sc_add(x2, pe2):
    return pl.kernel(
        _sc_body,
        out_type=jax.ShapeDtypeStruct((_B * _T, _D), jnp.float32),
        mesh=plsc.VectorSubcoreMesh(core_axis_name="c", subcore_axis_name="s"),
        scratch_types=[
            pltpu.VMEM((_R, _D), jnp.float32),
            pltpu.VMEM((_R, _D), jnp.float32),
            pltpu.VMEM((_B, _R, _D), jnp.float32),
            pltpu.VMEM((_B, _R, _D), jnp.float32),
            pltpu.SemaphoreType.DMA,
            pltpu.SemaphoreType.DMA,
            pltpu.SemaphoreType.DMA,
            pltpu.SemaphoreType.DMA,
        ],
    )(x2, pe2)


def kernel(x, pos_embed):
    B, T, D = x.shape
    out2 = _sc_add(x.reshape(B * T, D), pos_embed)
    return out2.reshape(B, T, D)


# speedup vs baseline: 2.9505x; 1.0046x over previous
import jax
import jax.numpy as jnp
from jax import lax
from jax.experimental import pallas as pl
from jax.experimental.pallas import tpu as pltpu
from jax.experimental.pallas import tpu_sc as plsc

_B = 4
_T = 4096
_D = 1024
_NW = 32
_R = 8
_NCH = _T // _NW // _R
_NV = _R * _D // 16
_NS = 3


def _sc_body(x2, pe2, out2,
             spe0, spe1, spe2, sxb0, sxb1, sxb2,
             si0, si1, si2, so0, so1, so2):
    spe = [spe0, spe1, spe2]
    sxb = [sxb0, sxb1, sxb2]
    si = [si0, si1, si2]
    so = [so0, so1, so2]

    wid = lax.axis_index("s") * 2 + lax.axis_index("c")
    base = wid * _NCH * _R

    def in_copies(j, s):
        row = base + j * _R
        cps = [pltpu.make_async_copy(pe2.at[pl.ds(row, _R)], spe[s], si[s])]
        for b in range(_B):
            cps.append(pltpu.make_async_copy(
                x2.at[pl.ds(b * _T + row, _R)], sxb[s].at[b], si[s]))
        return cps

    def out_copies(j, s):
        row = base + j * _R
        return [pltpu.make_async_copy(
            sxb[s].at[b], out2.at[pl.ds(b * _T + row, _R)], so[s])
            for b in range(_B)]

    for cp in in_copies(0, 0):
        cp.start()
    for cp in in_copies(1, 1):
        cp.start()
    for j in range(_NCH):
        s = j % _NS
        if j + 2 < _NCH:
            s2 = (j + 2) % _NS
            if j - 1 >= 0:
                for cp in out_copies(j - 1, s2):
                    cp.wait()
            for cp in in_copies(j + 2, s2):
                cp.start()
        for cp in in_copies(j, s):
            cp.wait()

        @plsc.parallel_loop(0, _NV, 1, unroll=8)
        def _(k):
            r = k // (_D // 16)
            sl = pl.ds((k % (_D // 16)) * 16, 16)
            pv = spe[s][r, sl]
            for b in range(_B):
                sxb[s][b, r, sl] = sxb[s][b, r, sl] + pv

        for cp in out_copies(j, s):
            cp.start()
    for j in (_NCH - 2, _NCH - 1):
        for cp in out_copies(j, j % _NS):
            cp.wait()


def _sc_add(x2, pe2):
    return pl.kernel(
        _sc_body,
        out_type=jax.ShapeDtypeStruct((_B * _T, _D), jnp.float32),
        mesh=plsc.VectorSubcoreMesh(core_axis_name="c", subcore_axis_name="s"),
        scratch_types=(
            [pltpu.VMEM((_R, _D), jnp.float32)] * _NS
            + [pltpu.VMEM((_B, _R, _D), jnp.float32)] * _NS
            + [pltpu.SemaphoreType.DMA] * (2 * _NS)
        ),
    )(x2, pe2)


def kernel(x, pos_embed):
    B, T, D = x.shape
    out2 = _sc_add(x.reshape(B * T, D), pos_embed)
    return out2.reshape(B, T, D)


# speedup vs baseline: 4.5360x; 1.5374x over previous
import jax
import jax.numpy as jnp
from jax.experimental import pallas as pl


_BT = 1024


def _add_block(x_ref, pe_ref, o_ref):
    o_ref[...] = x_ref[...] + pe_ref[...]


def kernel(x, pos_embed):
    B, T, D = x.shape
    grid = (T // _BT, B)
    return pl.pallas_call(
        _add_block,
        grid=grid,
        in_specs=[
            pl.BlockSpec((1, _BT, D), lambda i, b: (b, i, 0)),
            pl.BlockSpec((1, _BT, D), lambda i, b: (0, i, 0)),
        ],
        out_specs=pl.BlockSpec((1, _BT, D), lambda i, b: (b, i, 0)),
        out_shape=jax.ShapeDtypeStruct((B, T, D), x.dtype),
    )(x, pos_embed[None])


# speedup vs baseline: 4.8506x; 1.0694x over previous
import jax
import jax.numpy as jnp
from jax.experimental import pallas as pl


_BT = 2048


def _add_block(x_ref, pe_ref, o_ref):
    o_ref[...] = x_ref[...] + pe_ref[...]


def kernel(x, pos_embed):
    B, T, D = x.shape
    grid = (T // _BT, B)
    return pl.pallas_call(
        _add_block,
        grid=grid,
        in_specs=[
            pl.BlockSpec((1, _BT, D), lambda i, b: (b, i, 0)),
            pl.BlockSpec((1, _BT, D), lambda i, b: (0, i, 0)),
        ],
        out_specs=pl.BlockSpec((1, _BT, D), lambda i, b: (b, i, 0)),
        out_shape=jax.ShapeDtypeStruct((B, T, D), x.dtype),
    )(x, pos_embed[None])
